# Initial kernel scaffold; baseline (speedup 1.0000x reference)
import jax
import jax.numpy as jnp
from jax.experimental import pallas as pl


def kernel(inputs, gate_w, w1, w2, w3):
    raise NotImplementedError("write your pallas kernel here")


# speedup vs baseline: 1.5964x; 1.5964x over previous
import functools

import jax
import jax.numpy as jnp
from jax import lax
from jax.experimental import pallas as pl
from jax.experimental.pallas import tpu as pltpu
from jax.experimental.pallas import tpu_sc as plsc

E = 8
T = 2048
D = 1024
F = 1024
BM = 128
NBLK = (T * 2 + E * BM) // BM
NPAD = NBLK * BM

NC = 2
NS = 16
NW = NC * NS
TPW = T // NW
NG = 4
L = 16


def _routing_body(x_ref, gw_ref, pos0_ref, pos1_ref, p0_ref, p1_ref, bexp_ref):
    x = x_ref[...]
    gw = gw_ref[...]
    logits = lax.dot_general(x, gw, (((1,), (1,)), ((), ())),
                             preferred_element_type=jnp.float32)
    ids = lax.broadcasted_iota(jnp.int32, (T, E), 1)
    m0 = jnp.max(logits, axis=1, keepdims=True)
    e0 = jnp.min(jnp.where(logits == m0, ids, E), axis=1, keepdims=True)
    l2 = jnp.where(ids == e0, -1e30, logits)
    m1 = jnp.max(l2, axis=1, keepdims=True)
    e1 = jnp.min(jnp.where(l2 == m1, ids, E), axis=1, keepdims=True)
    p0 = 1.0 / (1.0 + jnp.exp(m1 - m0))
    oh0 = (ids == e0).astype(jnp.float32)
    oh1 = (ids == e1).astype(jnp.float32)
    cnt = oh0 + oh1

    C = 128
    NCH = T // C
    r = lax.broadcasted_iota(jnp.int32, (C, C), 0)
    c = lax.broadcasted_iota(jnp.int32, (C, C), 1)
    tri = (r > c).astype(jnp.float32)
    chunk_rank = []
    chunk_sums = []
    for k in range(NCH):
        blk = cnt[k * C:(k + 1) * C]
        chunk_rank.append(lax.dot_general(tri, blk, (((1,), (0,)), ((), ())),
                                          preferred_element_type=jnp.float32))
        chunk_sums.append(jnp.sum(blk, axis=0, keepdims=True))
    csum = jnp.concatenate(chunk_sums, axis=0)
    rN = lax.broadcasted_iota(jnp.int32, (NCH, NCH), 0)
    cN = lax.broadcasted_iota(jnp.int32, (NCH, NCH), 1)
    triN = (rN > cN).astype(jnp.float32)
    cpref = lax.dot_general(triN, csum, (((1,), (0,)), ((), ())),
                            preferred_element_type=jnp.float32)
    rank = jnp.concatenate(
        [chunk_rank[k] + cpref[k:k + 1] for k in range(NCH)], axis=0)

    counts = jnp.sum(cnt, axis=0, keepdims=True).astype(jnp.int32)
    padded = ((counts + (BM - 1)) >> 7) << 7
    r8 = lax.broadcasted_iota(jnp.int32, (E, E), 0)
    c8 = lax.broadcasted_iota(jnp.int32, (E, E), 1)
    tri8 = (r8 < c8).astype(jnp.float32)
    offs = lax.dot_general(padded.astype(jnp.float32), tri8,
                           (((1,), (0,)), ((), ())),
                           preferred_element_type=jnp.float32)
    dest = offs + rank
    pos0_ref[...] = jnp.sum(oh0 * dest, axis=1, keepdims=True).astype(jnp.int32)
    pos1_ref[...] = jnp.sum(oh1 * dest, axis=1, keepdims=True).astype(jnp.int32)
    p0_ref[...] = p0
    p1_ref[...] = 1.0 - p0

    ends = offs + padded.astype(jnp.float32)
    biota = (lax.broadcasted_iota(jnp.int32, (1, 128), 1) * BM).astype(jnp.float32)
    acc = jnp.zeros((1, 128), jnp.float32)
    for e in range(E):
        acc = acc + (ends[0, e] <= biota).astype(jnp.float32)
    bexp_ref[...] = jnp.minimum(acc, float(E - 1)).astype(jnp.int32)


def _routing(x, gate_w):
    return pl.pallas_call(
        _routing_body,
        out_shape=(
            jax.ShapeDtypeStruct((T, 1), jnp.int32),
            jax.ShapeDtypeStruct((T, 1), jnp.int32),
            jax.ShapeDtypeStruct((T, 1), jnp.float32),
            jax.ShapeDtypeStruct((T, 1), jnp.float32),
            jax.ShapeDtypeStruct((1, 128), jnp.int32),
        ),
    )(x, gate_w)


def _dispatch_body(x_hbm, pos0_hbm, pos1_hbm, out_hbm, idx0_v, idx1_v, xb_v,
                   sem):
    wid = lax.axis_index("s") * NC + lax.axis_index("c")
    base = wid * TPW
    pltpu.sync_copy(pos0_hbm.at[wid], idx0_v)
    pltpu.sync_copy(pos1_hbm.at[wid], idx1_v)
    pltpu.sync_copy(x_hbm.at[pl.ds(base, TPW)], xb_v)
    pltpu.async_copy(xb_v, out_hbm.at[idx0_v], sem).wait()
    pltpu.async_copy(xb_v, out_hbm.at[idx1_v], sem).wait()


@functools.partial(
    pl.kernel,
    out_type=jax.ShapeDtypeStruct((NPAD, D), jnp.float32),
    mesh=plsc.VectorSubcoreMesh(core_axis_name="c", subcore_axis_name="s"),
    scratch_types=[
        pltpu.VMEM((TPW,), jnp.int32),
        pltpu.VMEM((TPW,), jnp.int32),
        pltpu.VMEM((TPW, D), jnp.float32),
        pltpu.SemaphoreType.DMA,
    ],
)
def _dispatch(x_hbm, pos0_hbm, pos1_hbm, out_hbm, idx0_v, idx1_v, xb_v, sem):
    _dispatch_body(x_hbm, pos0_hbm, pos1_hbm, out_hbm, idx0_v, idx1_v, xb_v,
                   sem)


def _moe_mm_body(bexp_ref, x_ref, w1_ref, w3_ref, w2_ref, o_ref):
    x = x_ref[...]
    a = lax.dot_general(x, w1_ref[0], (((1,), (1,)), ((), ())),
                        preferred_element_type=jnp.float32)
    b = lax.dot_general(x, w3_ref[0], (((1,), (1,)), ((), ())),
                        preferred_element_type=jnp.float32)
    h = (a * (1.0 / (1.0 + jnp.exp(-a)))) * b
    o_ref[...] = lax.dot_general(h, w2_ref[0], (((1,), (0,)), ((), ())),
                                 preferred_element_type=jnp.float32)


def _moe_mm(bexp, x_sorted, w1, w2, w3):
    grid_spec = pltpu.PrefetchScalarGridSpec(
        num_scalar_prefetch=1,
        grid=(NBLK,),
        in_specs=[
            pl.BlockSpec((BM, D), lambda i, be: (i, 0)),
            pl.BlockSpec((1, F, D), lambda i, be: (be[i], 0, 0)),
            pl.BlockSpec((1, F, D), lambda i, be: (be[i], 0, 0)),
            pl.BlockSpec((1, F, D), lambda i, be: (be[i], 0, 0)),
        ],
        out_specs=pl.BlockSpec((BM, D), lambda i, be: (i, 0)),
    )
    return pl.pallas_call(
        _moe_mm_body,
        grid_spec=grid_spec,
        out_shape=jax.ShapeDtypeStruct((NPAD, D), jnp.float32),
        compiler_params=pltpu.CompilerParams(
            dimension_semantics=("arbitrary",)),
    )(bexp, x_sorted, w1, w3, w2)


def _combine_body(ey_hbm, pos0_hbm, pos1_hbm, p0_hbm, p1_hbm, out_hbm,
                  idx0_v, idx1_v, p0_v, p1_v, r0_v, r1_v, o_v, sem0, sem1):
    wid = lax.axis_index("s") * NC + lax.axis_index("c")
    base = wid * TPW
    pltpu.sync_copy(pos0_hbm.at[wid], idx0_v)
    pltpu.sync_copy(pos1_hbm.at[wid], idx1_v)
    pltpu.sync_copy(p0_hbm.at[pl.ds(base, TPW)], p0_v)
    pltpu.sync_copy(p1_hbm.at[pl.ds(base, TPW)], p1_v)
    for g in range(NG):
        c0 = pltpu.async_copy(ey_hbm.at[idx0_v.at[g]], r0_v, sem0)
        c1 = pltpu.async_copy(ey_hbm.at[idx1_v.at[g]], r1_v, sem1)
        c0.wait()
        c1.wait()
        p0g = p0_v[pl.ds(g * L, L)]
        p1g = p1_v[pl.ds(g * L, L)]

        dnums = lax.GatherDimensionNumbers(
            offset_dims=(), collapsed_slice_dims=(0,), start_index_map=(0,))

        def tok_body(j, _):
            bj = jnp.full((L, 1), j, jnp.int32)
            s0 = lax.gather(p0g, bj, dnums, (1,),
                            mode=lax.GatherScatterMode.PROMISE_IN_BOUNDS)
            s1 = lax.gather(p1g, bj, dnums, (1,),
                            mode=lax.GatherScatterMode.PROMISE_IN_BOUNDS)

            def ch_body(cc, __):
                sl = pl.ds(cc * L, L)
                o_v[j, sl] = s0 * r0_v[j, sl] + s1 * r1_v[j, sl]
                return __

            lax.fori_loop(0, D // L, ch_body, 0, unroll=4)
            return _

        lax.fori_loop(0, L, tok_body, 0)
        pltpu.sync_copy(o_v, out_hbm.at[pl.ds(base + g * L, L)])


@functools.partial(
    pl.kernel,
    out_type=jax.ShapeDtypeStruct((T, D), jnp.float32),
    mesh=plsc.VectorSubcoreMesh(core_axis_name="c", subcore_axis_name="s"),
    scratch_types=[
        pltpu.VMEM((NG, L), jnp.int32),
        pltpu.VMEM((NG, L), jnp.int32),
        pltpu.VMEM((TPW,), jnp.float32),
        pltpu.VMEM((TPW,), jnp.float32),
        pltpu.VMEM((L, D), jnp.float32),
        pltpu.VMEM((L, D), jnp.float32),
        pltpu.VMEM((L, D), jnp.float32),
        pltpu.SemaphoreType.DMA,
        pltpu.SemaphoreType.DMA,
    ],
)
def _combine(ey_hbm, pos0_hbm, pos1_hbm, p0_hbm, p1_hbm, out_hbm,
             idx0_v, idx1_v, p0_v, p1_v, r0_v, r1_v, o_v, sem0, sem1):
    _combine_body(ey_hbm, pos0_hbm, pos1_hbm, p0_hbm, p1_hbm, out_hbm,
                  idx0_v, idx1_v, p0_v, p1_v, r0_v, r1_v, o_v, sem0, sem1)


def kernel(inputs, gate_w, w1, w2, w3):
    pos0, pos1, p0, p1, bexp = _routing(inputs, gate_w)
    pos0 = pos0.reshape(NW, TPW)
    pos1 = pos1.reshape(NW, TPW)
    x_sorted = _dispatch(inputs, pos0, pos1)
    ey = _moe_mm(bexp.reshape(128)[:NBLK], x_sorted, w1, w2, w3)
    out = _combine(ey, pos0.reshape(NW, NG, L), pos1.reshape(NW, NG, L),
                   p0.reshape(T), p1.reshape(T))
    return out
